# baseline (speedup 1.0000x reference)
import jax
import jax.numpy as jnp
from jax.experimental import pallas as pl
from jax.experimental.pallas import tpu as pltpu

_TRANSFORMS = [jnp.tanh, jax.nn.relu, jax.nn.gelu, jax.nn.sigmoid]
_N = len(_TRANSFORMS)

_BLOCK_COLS = 2048


def _body(pair_ref, x_ref, o_ref):
    v = x_ref[...]
    branches = [
        (lambda v, i=i, j=j: _TRANSFORMS[j](_TRANSFORMS[i](v)))
        for i in range(_N)
        for j in range(_N)
    ]
    o_ref[...] = jax.lax.switch(pair_ref[0], branches, v)


def kernel(x):
    p = jnp.full((_N,), 1.0 / _N, jnp.float32)
    idx = jax.random.choice(
        jax.random.key(42), _N, shape=(2,), replace=False, p=p
    )
    pair = (idx[0] * _N + idx[1]).astype(jnp.int32).reshape((1,))

    rows, cols = x.shape
    grid = (cols // _BLOCK_COLS,)
    return pl.pallas_call(
        _body,
        grid_spec=pltpu.PrefetchScalarGridSpec(
            num_scalar_prefetch=1,
            grid=grid,
            in_specs=[
                pl.BlockSpec((rows, _BLOCK_COLS), lambda i, pair_ref: (0, i))
            ],
            out_specs=pl.BlockSpec(
                (rows, _BLOCK_COLS), lambda i, pair_ref: (0, i)
            ),
        ),
        out_shape=jax.ShapeDtypeStruct(x.shape, x.dtype),
    )(pair, x)


# speedup vs baseline: 2.7574x; 2.7574x over previous
import jax
import jax.numpy as jnp
from jax.experimental import pallas as pl
from jax.experimental.pallas import tpu as pltpu

_TRANSFORMS = [jnp.tanh, jax.nn.relu, jax.nn.gelu, jax.nn.sigmoid]
_N = len(_TRANSFORMS)

_BLOCK_COLS = 4096


def _make_call(i, j):
    def body(x_ref, o_ref):
        o_ref[...] = _TRANSFORMS[j](_TRANSFORMS[i](x_ref[...]))

    def call(x):
        rows, cols = x.shape
        return pl.pallas_call(
            body,
            grid=(cols // _BLOCK_COLS,),
            in_specs=[
                pl.BlockSpec((rows, _BLOCK_COLS), lambda g: (0, g))
            ],
            out_specs=pl.BlockSpec((rows, _BLOCK_COLS), lambda g: (0, g)),
            out_shape=jax.ShapeDtypeStruct(x.shape, x.dtype),
            compiler_params=pltpu.CompilerParams(
                dimension_semantics=("parallel",)
            ),
        )(x)

    return call


def kernel(x):
    p = jnp.full((_N,), 1.0 / _N, jnp.float32)
    idx = jax.random.choice(
        jax.random.key(42), _N, shape=(2,), replace=False, p=p
    )
    pair = (idx[0] * _N + idx[1]).astype(jnp.int32)
    branches = [_make_call(i, j) for i in range(_N) for j in range(_N)]
    return jax.lax.switch(pair, branches, x)


# speedup vs baseline: 3.2671x; 1.1848x over previous
import jax
import jax.numpy as jnp
import numpy as np
from jax.experimental import pallas as pl
from jax.experimental.pallas import tpu as pltpu

_TRANSFORMS = [jnp.tanh, jax.nn.relu, jax.nn.gelu, jax.nn.sigmoid]
_N = len(_TRANSFORMS)

_IDX = np.asarray(
    jax.random.choice(
        jax.random.key(42),
        _N,
        shape=(2,),
        replace=False,
        p=jnp.full((_N,), 1.0 / _N, jnp.float32),
    )
)
_I0, _I1 = int(_IDX[0]), int(_IDX[1])

_BLOCK_COLS = 4096


def _body(x_ref, o_ref):
    o_ref[...] = _TRANSFORMS[_I1](_TRANSFORMS[_I0](x_ref[...]))


def kernel(x):
    rows, cols = x.shape
    return pl.pallas_call(
        _body,
        grid=(cols // _BLOCK_COLS,),
        in_specs=[pl.BlockSpec((rows, _BLOCK_COLS), lambda g: (0, g))],
        out_specs=pl.BlockSpec((rows, _BLOCK_COLS), lambda g: (0, g)),
        out_shape=jax.ShapeDtypeStruct(x.shape, x.dtype),
        compiler_params=pltpu.CompilerParams(
            dimension_semantics=("parallel",)
        ),
    )(x)


# speedup vs baseline: 3.2936x; 1.0081x over previous
import jax
import jax.numpy as jnp
from jax.experimental import pallas as pl
from jax.experimental.pallas import tpu as pltpu

_TRANSFORMS = [jnp.tanh, jax.nn.relu, jax.nn.gelu, jax.nn.sigmoid]

_I0, _I1 = 1, 2

_BLOCK_ROWS = 16


def _body(x_ref, o_ref):
    o_ref[...] = _TRANSFORMS[_I1](_TRANSFORMS[_I0](x_ref[...]))


def kernel(x):
    rows, cols = x.shape
    return pl.pallas_call(
        _body,
        grid=(rows // _BLOCK_ROWS,),
        in_specs=[pl.BlockSpec((_BLOCK_ROWS, cols), lambda g: (g, 0))],
        out_specs=pl.BlockSpec((_BLOCK_ROWS, cols), lambda g: (g, 0)),
        out_shape=jax.ShapeDtypeStruct(x.shape, x.dtype),
        compiler_params=pltpu.CompilerParams(
            dimension_semantics=("parallel",)
        ),
    )(x)
